# baseline (speedup 1.0000x reference)
import functools

import jax
import jax.numpy as jnp
from jax import lax
from jax.experimental import pallas as pl
from jax.experimental.pallas import tpu as pltpu
from jax.experimental.pallas import tpu_sc as plsc

_NC = 2
_NS = 16
_NW = _NC * _NS


def _sc_gather(indices, table):
    (B,) = indices.shape
    _, D = table.shape
    bpw = B // _NW
    mesh = plsc.VectorSubcoreMesh(core_axis_name="c", subcore_axis_name="s")

    @functools.partial(
        pl.kernel,
        mesh=mesh,
        compiler_params=pltpu.CompilerParams(use_tc_tiling_on_sc=False),
        out_type=jax.ShapeDtypeStruct((B, D), table.dtype),
        scratch_types=[
            pltpu.VMEM((bpw,), jnp.int32),
            pltpu.VMEM((bpw, D), jnp.float32),
            pltpu.SemaphoreType.DMA,
        ],
    )
    def k(table_hbm, idx_hbm, out_hbm, idx_v, rows_v, sem):
        wid = lax.axis_index("s") * _NC + lax.axis_index("c")
        base = wid * bpw
        pltpu.sync_copy(idx_hbm.at[pl.ds(base, bpw)], idx_v)
        pltpu.async_copy(table_hbm.at[idx_v], rows_v, sem).wait()
        pltpu.sync_copy(rows_v, out_hbm.at[pl.ds(base, bpw)])

    return k(table, indices)


def _expmap_body(g_ref, o_ref):
    tx = g_ref[0]
    ty = g_ref[1]
    tz = g_ref[2]
    wx = g_ref[3]
    wy = g_ref[4]
    wz = g_ref[5]
    nrms = wx * wx + wy * wy + wz * wz
    ang = jnp.sqrt(jnp.maximum(nrms, 1e-4))
    inv = 1.0 / ang
    fac1 = inv * jnp.sin(ang)
    fac2 = inv * inv * (1.0 - jnp.cos(ang))
    xx = wx * wx
    yy = wy * wy
    zz = wz * wz
    xy = wx * wy
    xz = wx * wz
    yz = wy * wz
    o_ref[0] = 1.0 + fac2 * (xx - nrms)
    o_ref[1] = fac2 * xy - fac1 * wz
    o_ref[2] = fac2 * xz + fac1 * wy
    o_ref[3] = tx
    o_ref[4] = fac2 * xy + fac1 * wz
    o_ref[5] = 1.0 + fac2 * (yy - nrms)
    o_ref[6] = fac2 * yz - fac1 * wx
    o_ref[7] = ty
    o_ref[8] = fac2 * xz - fac1 * wy
    o_ref[9] = fac2 * yz + fac1 * wx
    o_ref[10] = 1.0 + fac2 * (zz - nrms)
    o_ref[11] = tz


def _expmap_tc(gt):
    _, R, C = gt.shape
    return pl.pallas_call(
        _expmap_body,
        out_shape=jax.ShapeDtypeStruct((12, R, C), jnp.float32),
    )(gt)


def kernel(indices, pose_adjustment):
    B = indices.shape[0]
    idx = indices.astype(jnp.int32)
    gathered = _sc_gather(idx, pose_adjustment)
    gt = gathered.T.reshape(6, B // 128, 128)
    out12 = _expmap_tc(gt)
    return out12.reshape(12, B).T.reshape(B, 3, 4)


# speedup vs baseline: 1.6058x; 1.6058x over previous
import functools

import jax
import jax.numpy as jnp
from jax import lax
from jax.experimental import pallas as pl
from jax.experimental.pallas import tpu as pltpu
from jax.experimental.pallas import tpu_sc as plsc

_NC = 2
_NS = 16
_NW = _NC * _NS


def _sc_gather_cols(indices, table_t):
    (B,) = indices.shape
    D, _ = table_t.shape
    bpw = B // _NW
    mesh = plsc.VectorSubcoreMesh(core_axis_name="c", subcore_axis_name="s")

    @functools.partial(
        pl.kernel,
        mesh=mesh,
        compiler_params=pltpu.CompilerParams(use_tc_tiling_on_sc=False),
        out_type=jax.ShapeDtypeStruct((D, B), table_t.dtype),
        scratch_types=[
            pltpu.VMEM((bpw,), jnp.int32),
            pltpu.VMEM((D, bpw), jnp.float32),
            pltpu.SemaphoreType.DMA,
        ],
    )
    def k(tab_hbm, idx_hbm, out_hbm, idx_v, cols_v, sem):
        wid = lax.axis_index("s") * _NC + lax.axis_index("c")
        base = wid * bpw
        pltpu.sync_copy(idx_hbm.at[pl.ds(base, bpw)], idx_v)
        copies = [
            pltpu.async_copy(tab_hbm.at[j].at[idx_v], cols_v.at[j], sem)
            for j in range(D)
        ]
        for c in copies:
            c.wait()
        for j in range(D):
            pltpu.sync_copy(cols_v.at[j], out_hbm.at[j, pl.ds(base, bpw)])

    return k(table_t, indices)


def _expmap_body(g_ref, o_ref):
    tx = g_ref[0]
    ty = g_ref[1]
    tz = g_ref[2]
    wx = g_ref[3]
    wy = g_ref[4]
    wz = g_ref[5]
    nrms = wx * wx + wy * wy + wz * wz
    ang = jnp.sqrt(jnp.maximum(nrms, 1e-4))
    inv = 1.0 / ang
    fac1 = inv * jnp.sin(ang)
    fac2 = inv * inv * (1.0 - jnp.cos(ang))
    xx = wx * wx
    yy = wy * wy
    zz = wz * wz
    xy = wx * wy
    xz = wx * wz
    yz = wy * wz
    o_ref[0] = 1.0 + fac2 * (xx - nrms)
    o_ref[1] = fac2 * xy - fac1 * wz
    o_ref[2] = fac2 * xz + fac1 * wy
    o_ref[3] = tx
    o_ref[4] = fac2 * xy + fac1 * wz
    o_ref[5] = 1.0 + fac2 * (yy - nrms)
    o_ref[6] = fac2 * yz - fac1 * wx
    o_ref[7] = ty
    o_ref[8] = fac2 * xz - fac1 * wy
    o_ref[9] = fac2 * yz + fac1 * wx
    o_ref[10] = 1.0 + fac2 * (zz - nrms)
    o_ref[11] = tz


def _expmap_tc(gt):
    _, R, C = gt.shape
    return pl.pallas_call(
        _expmap_body,
        out_shape=jax.ShapeDtypeStruct((12, R, C), jnp.float32),
    )(gt)


def kernel(indices, pose_adjustment):
    B = indices.shape[0]
    idx = indices.astype(jnp.int32)
    table_t = pose_adjustment.T
    cols = _sc_gather_cols(idx, table_t)
    out12 = _expmap_tc(cols.reshape(6, B // 128, 128))
    return out12.reshape(12, B).T.reshape(B, 3, 4)
